# Initial kernel scaffold; baseline (speedup 1.0000x reference)
import functools

import jax
import jax.numpy as jnp
from jax import lax
from jax.experimental import pallas as pl
from jax.experimental.pallas import tpu as pltpu
from jax.experimental.pallas import tpu_sc as plsc

N = 10000
E = 320000
D = 128

NC = 2
NS = 16
L = 16
NW = NC * NS
EPW = E // NW
C = 80
NCH = EPW // C
RPS = N // NS
DW = 16


def _zero_rows(buf, nrows, ncols):
    def body(i, carry):
        for j in range(ncols // L):
            buf[i, pl.ds(j * L, L)] = jnp.zeros((L,), jnp.float32)
        return carry
    lax.fori_loop(0, nrows, body, 0)


def _copy_rows_to(dst_ref, src_buf, base, total, bufrows):
    full, rem = divmod(total, bufrows)
    for t in range(full):
        pltpu.sync_copy(src_buf, dst_ref.at[pl.ds(base + t * bufrows, bufrows)])
    if rem:
        pltpu.sync_copy(src_buf.at[pl.ds(0, rem)],
                        dst_ref.at[pl.ds(base + full * bufrows, rem)])


def _sc_degrees(src, dst):
    mesh = plsc.VectorSubcoreMesh(core_axis_name="c", subcore_axis_name="s")

    @functools.partial(
        pl.kernel, mesh=mesh,
        out_type=[jax.ShapeDtypeStruct((NC, N, DW), jnp.float32),
                  jax.ShapeDtypeStruct((NC, N, DW), jnp.float32)],
        scratch_types=[
            pltpu.VMEM((C,), jnp.int32),
            pltpu.VMEM((C,), jnp.int32),
            pltpu.VMEM((C, DW), jnp.float32),
            pltpu.VMEM_SHARED((N, DW), jnp.float32),
            pltpu.VMEM_SHARED((N, DW), jnp.float32),
        ],
    )
    def k(src_hbm, dst_hbm, degs_out, degd_out, sidx, didx, ones, degs, degd):
        cid = lax.axis_index("c")
        sid = lax.axis_index("s")
        wid = sid * NC + cid
        base_r = sid * RPS

        _zero_rows(ones, C, DW)
        _copy_rows_to(degs, ones, base_r, RPS, C)
        _copy_rows_to(degd, ones, base_r, RPS, C)

        def fill(i, carry):
            ones[i, pl.ds(0, DW)] = jnp.full((DW,), 1.0, jnp.float32)
            return carry
        lax.fori_loop(0, C, fill, 0)
        plsc.subcore_barrier()

        ebase = wid * EPW

        def body(kk, carry):
            off = ebase + kk * C
            pltpu.sync_copy(src_hbm.at[pl.ds(off, C)], sidx)
            pltpu.sync_copy(dst_hbm.at[pl.ds(off, C)], didx)
            pltpu.sync_copy(ones, degs.at[sidx], add=True)
            pltpu.sync_copy(ones, degd.at[didx], add=True)
            return carry
        lax.fori_loop(0, NCH, body, 0)

        plsc.subcore_barrier()
        pltpu.sync_copy(degs.at[pl.ds(base_r, RPS)],
                        degs_out.at[cid, pl.ds(base_r, RPS)])
        pltpu.sync_copy(degd.at[pl.ds(base_r, RPS)],
                        degd_out.at[cid, pl.ds(base_r, RPS)])

    return k(src, dst)


def _sc_aggregate(h, src, dst, ew):
    mesh = plsc.VectorSubcoreMesh(core_axis_name="c", subcore_axis_name="s")

    @functools.partial(
        pl.kernel, mesh=mesh,
        out_type=jax.ShapeDtypeStruct((NC, N, D), jnp.float32),
        scratch_types=[
            pltpu.VMEM((C,), jnp.int32),
            pltpu.VMEM((C,), jnp.int32),
            pltpu.VMEM((C,), jnp.float32),
            pltpu.VMEM((C, D), jnp.float32),
            pltpu.VMEM_SHARED((N, D), jnp.float32),
            pltpu.SemaphoreType.DMA,
        ],
    )
    def k(h_hbm, src_hbm, dst_hbm, ew_hbm, out_hbm, sidx, didx, ewv, rows, acc, sem):
        cid = lax.axis_index("c")
        sid = lax.axis_index("s")
        wid = sid * NC + cid
        base_r = sid * RPS

        _zero_rows(rows, C, D)
        _copy_rows_to(acc, rows, base_r, RPS, C)
        plsc.subcore_barrier()

        ebase = wid * EPW

        def body(kk, carry):
            off = ebase + kk * C
            pltpu.sync_copy(src_hbm.at[pl.ds(off, C)], sidx)
            pltpu.sync_copy(dst_hbm.at[pl.ds(off, C)], didx)
            pltpu.sync_copy(ew_hbm.at[pl.ds(off, C)], ewv)
            pltpu.async_copy(h_hbm.at[sidx], rows, sem).wait()

            def scale(i, carry2):
                w = plsc.load_gather(ewv, [jnp.full((L,), i, jnp.int32)])
                for j in range(D // L):
                    rows[i, pl.ds(j * L, L)] = rows[i, pl.ds(j * L, L)] * w
                return carry2
            lax.fori_loop(0, C, scale, 0)

            pltpu.sync_copy(rows, acc.at[didx], add=True)
            return carry
        lax.fori_loop(0, NCH, body, 0)

        plsc.subcore_barrier()
        pltpu.sync_copy(acc.at[pl.ds(base_r, RPS)],
                        out_hbm.at[cid, pl.ds(base_r, RPS)])

    return k(h, src, dst, ew)


def _tc_norms(degs_p, degd_p):
    def body(ds_ref, dd_ref, ns_ref, nd_ref):
        s = jnp.sum(ds_ref[0] + ds_ref[1], axis=1, keepdims=True)
        d = jnp.sum(dd_ref[0] + dd_ref[1], axis=1, keepdims=True)
        ns_ref[...] = 4.0 * lax.rsqrt(jnp.maximum(s, 16.0))
        nd_ref[...] = 4.0 * lax.rsqrt(jnp.maximum(d, 16.0))

    return pl.pallas_call(
        body,
        out_shape=[jax.ShapeDtypeStruct((N, 1), jnp.float32),
                   jax.ShapeDtypeStruct((N, 1), jnp.float32)],
    )(degs_p, degd_p)


_R = 2000


def _tc_mm_scale(x, W, ns):
    def body(x_ref, w_ref, ns_ref, o_ref):
        o_ref[...] = jnp.dot(x_ref[...], w_ref[...],
                             preferred_element_type=jnp.float32) * ns_ref[...]

    return pl.pallas_call(
        body,
        grid=(N // _R,),
        in_specs=[pl.BlockSpec((_R, D), lambda i: (i, 0)),
                  pl.BlockSpec((D, D), lambda i: (0, 0)),
                  pl.BlockSpec((_R, 1), lambda i: (i, 0))],
        out_specs=pl.BlockSpec((_R, D), lambda i: (i, 0)),
        out_shape=jax.ShapeDtypeStruct((N, D), jnp.float32),
    )(x, W, ns)


def _tc_combine_mm(pa, pb, nd, b, W, ns):
    def body(pa_ref, pb_ref, nd_ref, b_ref, w_ref, ns_ref, o_ref):
        x = jnp.maximum((pa_ref[...] + pb_ref[...]) * nd_ref[...] + b_ref[...],
                        0.0)
        o_ref[...] = jnp.dot(x, w_ref[...],
                             preferred_element_type=jnp.float32) * ns_ref[...]

    return pl.pallas_call(
        body,
        grid=(N // _R,),
        in_specs=[pl.BlockSpec((_R, D), lambda i: (i, 0)),
                  pl.BlockSpec((_R, D), lambda i: (i, 0)),
                  pl.BlockSpec((_R, 1), lambda i: (i, 0)),
                  pl.BlockSpec((1, D), lambda i: (0, 0)),
                  pl.BlockSpec((D, D), lambda i: (0, 0)),
                  pl.BlockSpec((_R, 1), lambda i: (i, 0))],
        out_specs=pl.BlockSpec((_R, D), lambda i: (i, 0)),
        out_shape=jax.ShapeDtypeStruct((N, D), jnp.float32),
    )(pa, pb, nd, b, W, ns)


def _tc_final(pa, pb, nd, b):
    def body(pa_ref, pb_ref, nd_ref, b_ref, o_ref):
        o_ref[...] = (pa_ref[...] + pb_ref[...]) * nd_ref[...] + b_ref[...]

    return pl.pallas_call(
        body,
        grid=(N // _R,),
        in_specs=[pl.BlockSpec((_R, D), lambda i: (i, 0)),
                  pl.BlockSpec((_R, D), lambda i: (i, 0)),
                  pl.BlockSpec((_R, 1), lambda i: (i, 0)),
                  pl.BlockSpec((1, D), lambda i: (0, 0))],
        out_specs=pl.BlockSpec((_R, D), lambda i: (i, 0)),
        out_shape=jax.ShapeDtypeStruct((N, D), jnp.float32),
    )(pa, pb, nd, b)


def kernel(features, edge_index, edge_weights, W0, b0, W1, b1, Wp, bp):
    src = edge_index[0]
    dst = edge_index[1]

    degs_p, degd_p = _sc_degrees(src, dst)
    ns, nd = _tc_norms(degs_p, degd_p)

    b0r = b0.reshape(1, D)
    b1r = b1.reshape(1, D)
    bpr = bp.reshape(1, D)

    h = _tc_mm_scale(features, W0, ns)
    p = _sc_aggregate(h, src, dst, edge_weights)
    h = _tc_combine_mm(p[0], p[1], nd, b0r, W1, ns)
    p = _sc_aggregate(h, src, dst, edge_weights)
    h = _tc_combine_mm(p[0], p[1], nd, b1r, Wp, ns)
    p = _sc_aggregate(h, src, dst, edge_weights)
    return _tc_final(p[0], p[1], nd, bpr)


# speedup vs baseline: 5.0176x; 5.0176x over previous
import functools

import jax
import jax.numpy as jnp
from jax import lax
from jax.experimental import pallas as pl
from jax.experimental.pallas import tpu as pltpu
from jax.experimental.pallas import tpu_sc as plsc

N = 10000
E = 320000
D = 128

NC = 2
NS = 16
L = 16
NW = NC * NS
EPW = E // NW
C = 80
NCH = EPW // C
RB = 624
TAIL = N - NS * RB
DW = 16


def _zero_rows(buf, nrows, ncols):
    def body(i, carry):
        for j in range(ncols // L):
            buf[i, pl.ds(j * L, L)] = jnp.zeros((L,), jnp.float32)
        return carry
    lax.fori_loop(0, nrows, body, 0)


def _copy_rows_to(dst_ref, src_buf, base, total, bufrows):
    full, rem = divmod(total, bufrows)
    for t in range(full):
        pltpu.sync_copy(src_buf, dst_ref.at[pl.ds(base + t * bufrows, bufrows)])
    if rem:
        pltpu.sync_copy(src_buf.at[pl.ds(0, rem)],
                        dst_ref.at[pl.ds(base + full * bufrows, rem)])


def _sc_degrees(src, dst):
    mesh = plsc.VectorSubcoreMesh(core_axis_name="c", subcore_axis_name="s")

    @functools.partial(
        pl.kernel, mesh=mesh,
        out_type=jax.ShapeDtypeStruct((NC, N, D), jnp.float32),
        scratch_types=[
            pltpu.VMEM((C,), jnp.int32),
            pltpu.VMEM((C,), jnp.int32),
            pltpu.VMEM((C, D), jnp.float32),
            pltpu.VMEM((C, D), jnp.float32),
            pltpu.VMEM_SHARED((N, D), jnp.float32),
        ],
    )
    def k(src_hbm, dst_hbm, out_hbm, sidx, didx, ones_s, ones_d, tab):
        cid = lax.axis_index("c")
        sid = lax.axis_index("s")
        wid = sid * NC + cid
        base_r = sid * RB

        _zero_rows(ones_s, C, D)
        _copy_rows_to(tab, ones_s, base_r, RB, C)

        @pl.when(sid == NS - 1)
        def _():
            _copy_rows_to(tab, ones_s, NS * RB, TAIL, C)

        half = D // (2 * L)

        def fill(i, carry):
            for j in range(D // L):
                sv = 1.0 if j < half else 0.0
                ones_s[i, pl.ds(j * L, L)] = jnp.full((L,), sv, jnp.float32)
                ones_d[i, pl.ds(j * L, L)] = jnp.full((L,), 1.0 - sv, jnp.float32)
            return carry
        lax.fori_loop(0, C, fill, 0)
        plsc.subcore_barrier()

        ebase = wid * EPW

        def body(kk, carry):
            off = ebase + kk * C
            pltpu.sync_copy(src_hbm.at[pl.ds(off, C)], sidx)
            pltpu.sync_copy(dst_hbm.at[pl.ds(off, C)], didx)
            pltpu.sync_copy(ones_s, tab.at[sidx], add=True)
            pltpu.sync_copy(ones_d, tab.at[didx], add=True)
            return carry
        lax.fori_loop(0, NCH, body, 0)

        plsc.subcore_barrier()
        pltpu.sync_copy(tab.at[pl.ds(base_r, RB)],
                        out_hbm.at[cid, pl.ds(base_r, RB)])

        @pl.when(sid == NS - 1)
        def _():
            pltpu.sync_copy(tab.at[pl.ds(NS * RB, TAIL)],
                            out_hbm.at[cid, pl.ds(NS * RB, TAIL)])

    return k(src, dst)


def _sc_aggregate(h, src, dst, ew):
    mesh = plsc.VectorSubcoreMesh(core_axis_name="c", subcore_axis_name="s")

    @functools.partial(
        pl.kernel, mesh=mesh,
        out_type=jax.ShapeDtypeStruct((NC, N, D), jnp.float32),
        scratch_types=[
            pltpu.VMEM((C,), jnp.int32),
            pltpu.VMEM((C,), jnp.int32),
            pltpu.VMEM((C,), jnp.float32),
            pltpu.VMEM((C, D), jnp.float32),
            pltpu.VMEM_SHARED((N, D), jnp.float32),
            pltpu.SemaphoreType.DMA,
        ],
    )
    def k(h_hbm, src_hbm, dst_hbm, ew_hbm, out_hbm, sidx, didx, ewv, rows, acc, sem):
        cid = lax.axis_index("c")
        sid = lax.axis_index("s")
        wid = sid * NC + cid
        base_r = sid * RB

        _zero_rows(rows, C, D)
        _copy_rows_to(acc, rows, base_r, RB, C)

        @pl.when(sid == NS - 1)
        def _():
            _copy_rows_to(acc, rows, NS * RB, TAIL, C)

        plsc.subcore_barrier()

        ebase = wid * EPW

        def body(kk, carry):
            off = ebase + kk * C
            pltpu.sync_copy(src_hbm.at[pl.ds(off, C)], sidx)
            pltpu.sync_copy(dst_hbm.at[pl.ds(off, C)], didx)
            pltpu.sync_copy(ew_hbm.at[pl.ds(off, C)], ewv)
            pltpu.async_copy(h_hbm.at[sidx], rows, sem).wait()

            def scale(g, carry2):
                wv = ewv[pl.ds(g * L, L)]
                for k in range(L):
                    i = g * L + k
                    w = wv[k]
                    for j in range(D // L):
                        rows[i, pl.ds(j * L, L)] = rows[i, pl.ds(j * L, L)] * w
                return carry2
            lax.fori_loop(0, C // L, scale, 0)

            pltpu.sync_copy(rows, acc.at[didx], add=True)
            return carry
        lax.fori_loop(0, NCH, body, 0)

        plsc.subcore_barrier()
        pltpu.sync_copy(acc.at[pl.ds(base_r, RB)],
                        out_hbm.at[cid, pl.ds(base_r, RB)])

        @pl.when(sid == NS - 1)
        def _():
            pltpu.sync_copy(acc.at[pl.ds(NS * RB, TAIL)],
                            out_hbm.at[cid, pl.ds(NS * RB, TAIL)])

    return k(h, src, dst, ew)


def _tc_norms(p0, p1):
    def body(a_ref, b_ref, ns_ref, nd_ref):
        t = a_ref[...] + b_ref[...]
        ns_ref[...] = lax.rsqrt(jnp.maximum(t[:, 0:1], 1.0))
        nd_ref[...] = lax.rsqrt(jnp.maximum(t[:, D // 2:D // 2 + 1], 1.0))

    return pl.pallas_call(
        body,
        grid=(N // _R,),
        in_specs=[pl.BlockSpec((_R, D), lambda i: (i, 0)),
                  pl.BlockSpec((_R, D), lambda i: (i, 0))],
        out_specs=[pl.BlockSpec((_R, 1), lambda i: (i, 0)),
                   pl.BlockSpec((_R, 1), lambda i: (i, 0))],
        out_shape=[jax.ShapeDtypeStruct((N, 1), jnp.float32),
                   jax.ShapeDtypeStruct((N, 1), jnp.float32)],
    )(p0, p1)


_R = 2000


def _tc_mm_scale(x, W, ns):
    def body(x_ref, w_ref, ns_ref, o_ref):
        o_ref[...] = jnp.dot(x_ref[...], w_ref[...],
                             preferred_element_type=jnp.float32) * ns_ref[...]

    return pl.pallas_call(
        body,
        grid=(N // _R,),
        in_specs=[pl.BlockSpec((_R, D), lambda i: (i, 0)),
                  pl.BlockSpec((D, D), lambda i: (0, 0)),
                  pl.BlockSpec((_R, 1), lambda i: (i, 0))],
        out_specs=pl.BlockSpec((_R, D), lambda i: (i, 0)),
        out_shape=jax.ShapeDtypeStruct((N, D), jnp.float32),
    )(x, W, ns)


def _tc_combine_mm(pa, pb, nd, b, W, ns):
    def body(pa_ref, pb_ref, nd_ref, b_ref, w_ref, ns_ref, o_ref):
        x = jnp.maximum((pa_ref[...] + pb_ref[...]) * nd_ref[...] + b_ref[...],
                        0.0)
        o_ref[...] = jnp.dot(x, w_ref[...],
                             preferred_element_type=jnp.float32) * ns_ref[...]

    return pl.pallas_call(
        body,
        grid=(N // _R,),
        in_specs=[pl.BlockSpec((_R, D), lambda i: (i, 0)),
                  pl.BlockSpec((_R, D), lambda i: (i, 0)),
                  pl.BlockSpec((_R, 1), lambda i: (i, 0)),
                  pl.BlockSpec((1, D), lambda i: (0, 0)),
                  pl.BlockSpec((D, D), lambda i: (0, 0)),
                  pl.BlockSpec((_R, 1), lambda i: (i, 0))],
        out_specs=pl.BlockSpec((_R, D), lambda i: (i, 0)),
        out_shape=jax.ShapeDtypeStruct((N, D), jnp.float32),
    )(pa, pb, nd, b, W, ns)


def _tc_final(pa, pb, nd, b):
    def body(pa_ref, pb_ref, nd_ref, b_ref, o_ref):
        o_ref[...] = (pa_ref[...] + pb_ref[...]) * nd_ref[...] + b_ref[...]

    return pl.pallas_call(
        body,
        grid=(N // _R,),
        in_specs=[pl.BlockSpec((_R, D), lambda i: (i, 0)),
                  pl.BlockSpec((_R, D), lambda i: (i, 0)),
                  pl.BlockSpec((_R, 1), lambda i: (i, 0)),
                  pl.BlockSpec((1, D), lambda i: (0, 0))],
        out_specs=pl.BlockSpec((_R, D), lambda i: (i, 0)),
        out_shape=jax.ShapeDtypeStruct((N, D), jnp.float32),
    )(pa, pb, nd, b)


def kernel(features, edge_index, edge_weights, W0, b0, W1, b1, Wp, bp):
    src = edge_index[0]
    dst = edge_index[1]

    deg_p = _sc_degrees(src, dst)
    ns, nd = _tc_norms(deg_p[0], deg_p[1])

    b0r = b0.reshape(1, D)
    b1r = b1.reshape(1, D)
    bpr = bp.reshape(1, D)

    h = _tc_mm_scale(features, W0, ns)
    p = _sc_aggregate(h, src, dst, edge_weights)
    h = _tc_combine_mm(p[0], p[1], nd, b0r, W1, ns)
    p = _sc_aggregate(h, src, dst, edge_weights)
    h = _tc_combine_mm(p[0], p[1], nd, b1r, Wp, ns)
    p = _sc_aggregate(h, src, dst, edge_weights)
    return _tc_final(p[0], p[1], nd, bpr)
